# baseline (speedup 1.0000x reference)
import jax
import jax.numpy as jnp
from jax import lax
from jax.experimental import pallas as pl
from jax.experimental.pallas import tpu as pltpu
from jax.experimental.pallas import tpu_sc as plsc

NC = 2
NS = 16
NW = NC * NS
TZ = 128
LANES = 16
NOUT = 192
N1 = 96
N2 = 96
OPW = NOUT // NW
CBW = 4096
SC_Z = 4096


def _sc_body(f1s_hbm, f2s_hbm, i_hbm, j_hbm, v_hbm, rp_hbm, out_hbm,
             f1_v, f2_v, out_v, i_v, j_v, v_v, rp_v):
    w = lax.axis_index("s") * NC + lax.axis_index("c")
    base = w * CBW
    pltpu.sync_copy(i_hbm.at[pl.ds(base, CBW)], i_v)
    pltpu.sync_copy(j_hbm.at[pl.ds(base, CBW)], j_v)
    pltpu.sync_copy(v_hbm.at[pl.ds(base, CBW)], v_v)
    pltpu.sync_copy(rp_hbm, rp_v)
    ptr = rp_v[pl.ds(w * OPW, LANES)]

    nzt = SC_Z // TZ

    def zt_body(zt, carry):
        zsl = pl.ds(zt * TZ, TZ)
        pltpu.sync_copy(f1s_hbm.at[:, zsl], f1_v)
        pltpu.sync_copy(f2s_hbm.at[:, zsl], f2_v)
        for t in range(OPW):
            g_lo = (ptr[t] - base) // LANES
            g_hi = (ptr[t + 1] - base) // LANES
            zero = jnp.zeros((LANES,), jnp.float32)
            acc0 = (zero,) * (TZ // LANES)

            def seg(g, acc, _t=t):
                nsl = pl.ds(g * LANES, LANES)
                iv = i_v[nsl]
                jv = j_v[nsl]
                vv = v_v[nsl]
                new = list(acc)
                for u in range(LANES):
                    ii = iv[u]
                    jj = jv[u]
                    vt = vv[u]
                    for r in range(TZ // LANES):
                        sl = pl.ds(r * LANES, LANES)
                        new[r] = new[r] + f1_v[ii, sl] * f2_v[jj, sl] * vt
                return tuple(new)

            accs = lax.fori_loop(g_lo, g_hi, seg, acc0)
            for r in range(TZ // LANES):
                out_v[t, pl.ds(r * LANES, LANES)] = accs[r]
        pltpu.sync_copy(out_v.at[pl.ds(0, OPW)], out_hbm.at[w, :, zsl])
        return carry

    lax.fori_loop(0, nzt, zt_body, 0)


def _sc_call(f1s, f2s, i_arr, j_arr, v_arr, rowptr):
    return pl.kernel(
        _sc_body,
        out_type=jax.ShapeDtypeStruct((NW, OPW, SC_Z), jnp.float32),
        mesh=plsc.VectorSubcoreMesh(
            core_axis_name="c", subcore_axis_name="s",
            num_cores=NC, num_subcores=NS),
        scratch_types=[
            pltpu.VMEM((N1, TZ), jnp.float32),
            pltpu.VMEM((N2, TZ), jnp.float32),
            pltpu.VMEM((8, TZ), jnp.float32),
            pltpu.VMEM((CBW,), jnp.int32),
            pltpu.VMEM((CBW,), jnp.int32),
            pltpu.VMEM((CBW,), jnp.float32),
            pltpu.VMEM((208,), jnp.int32),
        ],
    )(f1s, f2s, i_arr, j_arr, v_arr, rowptr)


def _sc_preprocess(mixing_matrix):
    nout, nk = mixing_matrix.shape
    flat = mixing_matrix.reshape(-1)
    nzmask = flat != 0.0
    counts = jnp.sum(nzmask.reshape(nout, nk), axis=1).astype(jnp.int32)
    padded = ((counts + LANES - 1) // LANES) * LANES
    cps = jnp.cumsum(padded)
    excl = cps - padded
    blk = jnp.arange(nout, dtype=jnp.int32) // OPW
    blk_start = excl[blk * OPW]
    rowptr = blk * CBW + excl - blk_start
    last_end = (NW - 1) * CBW + cps[nout - 1] - excl[(NW - 1) * OPW]
    rowptr_full = jnp.concatenate(
        [rowptr, jnp.full((208 - nout,), last_end, jnp.int32)])

    cap = CBW * NW
    idx = jnp.nonzero(nzmask, size=cap, fill_value=0)[0].astype(jnp.int32)
    count_tot = jnp.sum(counts)
    valid = jnp.arange(cap, dtype=jnp.int32) < count_tot
    vals = jnp.where(valid, flat[idx], 0.0)
    o_id = idx // nk
    k_id = idx % nk
    i_id = k_id // N2
    j_id = k_id % N2
    excl_orig = jnp.cumsum(counts) - counts
    rank = jnp.arange(cap, dtype=jnp.int32) - excl_orig[o_id]
    pos = rowptr[o_id] + rank
    pos = jnp.where(valid & (pos < (blk[o_id] + 1) * CBW), pos, cap)
    i_arr = jnp.zeros((cap,), jnp.int32).at[pos].set(i_id, mode='drop')
    j_arr = jnp.zeros((cap,), jnp.int32).at[pos].set(j_id, mode='drop')
    v_arr = jnp.zeros((cap,), jnp.float32).at[pos].set(vals, mode='drop')
    return i_arr, j_arr, v_arr, rowptr_full


def _tc_body(f1_ref, f2_ref, w_ref, o_ref):
    f1t = f1_ref[...].astype(jnp.bfloat16).T
    f2t = f2_ref[...].astype(jnp.bfloat16).T
    n1, bz = f1t.shape
    n2 = f2t.shape[0]
    big = (f1t[:, None, :] * f2t[None, :, :]).reshape(n1 * n2, bz)
    w = w_ref[...].astype(jnp.bfloat16)
    o_ref[...] = jnp.dot(w, big, preferred_element_type=jnp.float32)


def _tc_call(features_1, features_2, mixing_matrix, bz):
    z, n1 = features_1.shape
    n2 = features_2.shape[1]
    n_out = mixing_matrix.shape[0]
    return pl.pallas_call(
        _tc_body,
        grid=(z // bz,),
        in_specs=[
            pl.BlockSpec((bz, n1), lambda g: (g, 0)),
            pl.BlockSpec((bz, n2), lambda g: (g, 0)),
            pl.BlockSpec((n_out, n1 * n2), lambda g: (0, 0)),
        ],
        out_specs=pl.BlockSpec((n_out, bz), lambda g: (0, g)),
        out_shape=jax.ShapeDtypeStruct((n_out, z), jnp.float32),
    )(features_1, features_2, mixing_matrix)


def kernel(features_1, features_2, mixing_matrix):
    z, _ = features_1.shape
    i_arr, j_arr, v_arr, rowptr = _sc_preprocess(mixing_matrix)
    f1s = features_1[z - SC_Z:].T
    f2s = features_2[z - SC_Z:].T
    outs = _sc_call(f1s, f2s, i_arr, j_arr, v_arr, rowptr).reshape(NOUT, SC_Z)
    parts = [outs]
    if SC_Z < z:
        outt = _tc_call(features_1[:z - SC_Z], features_2[:z - SC_Z],
                        mixing_matrix, 512)
        parts = [outt, outs]
    return jnp.concatenate(parts, axis=1).T


# speedup vs baseline: 402.0848x; 402.0848x over previous
import jax
import jax.numpy as jnp
from jax.experimental import pallas as pl


def _body(f1_ref, f2_ref, w_ref, o_ref):
    f1t = f1_ref[...].astype(jnp.bfloat16).T
    f2t = f2_ref[...].astype(jnp.bfloat16).T
    n1, bz = f1t.shape
    n2 = f2t.shape[0]
    big = (f1t[:, None, :] * f2t[None, :, :]).reshape(n1 * n2, bz)
    w = w_ref[...].astype(jnp.bfloat16)
    o_ref[...] = jnp.dot(w, big, preferred_element_type=jnp.float32)


def kernel(features_1, features_2, mixing_matrix):
    z, n1 = features_1.shape
    n2 = features_2.shape[1]
    n_out = mixing_matrix.shape[0]
    bz = 1024
    outt = pl.pallas_call(
        _body,
        grid=(z // bz,),
        in_specs=[
            pl.BlockSpec((bz, n1), lambda g: (g, 0)),
            pl.BlockSpec((bz, n2), lambda g: (g, 0)),
            pl.BlockSpec((n_out, n1 * n2), lambda g: (0, 0)),
        ],
        out_specs=pl.BlockSpec((n_out, bz), lambda g: (0, g)),
        out_shape=jax.ShapeDtypeStruct((n_out, z), jnp.float32),
    )(features_1, features_2, mixing_matrix)
    return outt.T


# speedup vs baseline: 407.2229x; 1.0128x over previous
import jax
import jax.numpy as jnp
from jax.experimental import pallas as pl


def _body(f1_ref, f2_ref, w_ref, o_ref):
    f1t = f1_ref[...].astype(jnp.bfloat16).T
    f2t = f2_ref[...].astype(jnp.bfloat16).T
    n1, bz = f1t.shape
    n2 = f2t.shape[0]
    big = (f1t[:, None, :] * f2t[None, :, :]).reshape(n1 * n2, bz)
    w = w_ref[...].astype(jnp.bfloat16)
    o_ref[...] = jnp.dot(w, big, preferred_element_type=jnp.float32)


def kernel(features_1, features_2, mixing_matrix):
    z, n1 = features_1.shape
    n2 = features_2.shape[1]
    n_out = mixing_matrix.shape[0]
    bz = 2048
    outt = pl.pallas_call(
        _body,
        grid=(z // bz,),
        in_specs=[
            pl.BlockSpec((bz, n1), lambda g: (g, 0)),
            pl.BlockSpec((bz, n2), lambda g: (g, 0)),
            pl.BlockSpec((n_out, n1 * n2), lambda g: (0, 0)),
        ],
        out_specs=pl.BlockSpec((n_out, bz), lambda g: (0, g)),
        out_shape=jax.ShapeDtypeStruct((n_out, z), jnp.float32),
    )(features_1, features_2, mixing_matrix)
    return outt.T
